# baseline (speedup 1.0000x reference)
import functools

import jax
import jax.numpy as jnp
from jax import lax
from jax.experimental import pallas as pl
from jax.experimental.pallas import tpu as pltpu
from jax.experimental.pallas import tpu_sc as plsc

_LANES = 16
_NW = 32
_GCH = 128
_NCHUNKS = 4
_BM = 512
_NBUF = 4


def _sc_hash_gather(tokens_flat, embed_weight, seq_len, chunk_off, mc):
    vocab, dim = embed_weight.shape
    tw = mc // _NW
    nch = tw // _GCH
    mod = jnp.int32(vocab - 1)

    mesh = plsc.VectorSubcoreMesh(core_axis_name="c", subcore_axis_name="s")

    @functools.partial(
        pl.kernel,
        out_type=jax.ShapeDtypeStruct((_NW, nch, _GCH, dim), jnp.float32),
        mesh=mesh,
        scratch_types=[
            pltpu.VMEM((tw + 8,), jnp.int32),
            pltpu.VMEM((nch, _GCH), jnp.int32),
            pltpu.VMEM((nch, _GCH, dim), jnp.float32),
            pltpu.SemaphoreType.DMA((nch,)),
            pltpu.SemaphoreType.DMA,
        ],
    )
    def k(tok_hbm, table_hbm, out_hbm, tok_v, idx_v, rows_v, gsems, wsem):
        wid = lax.axis_index("s") * 2 + lax.axis_index("c")
        base = chunk_off + wid * tw

        if chunk_off == 0:
            @pl.when(wid == 0)
            def _():
                pltpu.sync_copy(tok_hbm.at[pl.ds(0, tw)],
                                tok_v.at[pl.ds(8, tw)])

            @pl.when(wid != 0)
            def _():
                pltpu.sync_copy(tok_hbm.at[pl.ds(base - 8, tw + 8)], tok_v)
        else:
            pltpu.sync_copy(tok_hbm.at[pl.ds(base - 8, tw + 8)], tok_v)

        inv_mod = jnp.float32(1.0 / float(vocab - 1))
        for c in range(nch):
            for j in range(_GCH // _LANES):
                off = c * _GCH + j * _LANES
                prev = tok_v[pl.ds(off + 7, _LANES)]
                cur = tok_v[pl.ds(off + 8, _LANES)]
                x = lax.bitwise_xor(jnp.int32(36313) * cur,
                                    jnp.int32(27191) * prev)
                q = (x.astype(jnp.float32) * inv_mod).astype(jnp.int32)
                r = x - q * mod
                neg = lax.shift_right_arithmetic(r, 31)
                r = r - neg * mod
                t = r - mod
                tneg = lax.shift_right_arithmetic(t, 31)
                r = t - tneg * mod
                idx_v[c, pl.ds(j * _LANES, _LANES)] = r

        @pl.when((base % seq_len) == 0)
        def _():
            first = idx_v[0, pl.ds(0, _LANES)]
            keep = jnp.minimum(lax.iota(jnp.int32, _LANES), 1)
            idx_v[0, pl.ds(0, _LANES)] = first * keep + mod * (1 - keep)

        gathers = [
            pltpu.make_async_copy(table_hbm.at[idx_v.at[c]], rows_v.at[c],
                                  gsems.at[c])
            for c in range(nch)
        ]
        writes = [
            pltpu.make_async_copy(rows_v.at[c], out_hbm.at[wid, c], wsem)
            for c in range(nch)
        ]
        for g in gathers:
            g.start()
        for c in range(nch):
            gathers[c].wait()
            writes[c].start()
        for w in writes:
            w.wait()

    return k(tokens_flat, embed_weight).reshape(mc, dim)


def _tc_project_chunk(h_c, proj_weight, scale, total_m, block_off, prev):
    mc, kdim = h_c.shape
    n = proj_weight.shape[0]
    bm = _BM
    ng = mc // bm
    nbuf = min(_NBUF, ng)
    row_off = block_off * bm

    def body(s_ref, h_ref, w_ref, *rest):
        o_hbm, acc_ref, sems = rest[-3], rest[-2], rest[-1]
        i = pl.program_id(0)
        slot = lax.rem(i, nbuf)

        @pl.when(i >= nbuf)
        def _():
            j = i - nbuf
            pltpu.make_async_copy(
                acc_ref.at[slot],
                o_hbm.at[pl.ds(row_off + j * bm, bm)],
                sems.at[slot]).wait()

        acc = lax.dot_general(
            h_ref[...], w_ref[...], (((1,), (1,)), ((), ())),
            preferred_element_type=jnp.float32)
        acc_ref[slot] = acc * s_ref[0]
        pltpu.make_async_copy(
            acc_ref.at[slot],
            o_hbm.at[pl.ds(row_off + i * bm, bm)],
            sems.at[slot]).start()

        @pl.when(i == ng - 1)
        def _():
            for it in range(ng - nbuf, ng):
                pltpu.make_async_copy(
                    acc_ref.at[it % nbuf],
                    o_hbm.at[pl.ds(row_off + it * bm, bm)],
                    sems.at[it % nbuf]).wait()

    in_specs = [
        pl.BlockSpec(memory_space=pltpu.SMEM),
        pl.BlockSpec((bm, kdim), lambda i: (i, 0)),
        pl.BlockSpec((n, kdim), lambda i: (0, 0)),
    ]
    args = [scale.reshape(1), h_c, proj_weight]
    aliases = {}
    if prev is not None:
        in_specs.append(pl.BlockSpec(memory_space=pl.ANY))
        args.append(prev)
        aliases = {3: 0}

    return pl.pallas_call(
        body,
        grid=(ng,),
        in_specs=in_specs,
        out_specs=pl.BlockSpec(memory_space=pl.ANY),
        out_shape=jax.ShapeDtypeStruct((total_m, n), jnp.float32),
        input_output_aliases=aliases,
        scratch_shapes=[
            pltpu.VMEM((nbuf, bm, n), jnp.float32),
            pltpu.SemaphoreType.DMA((nbuf,)),
        ],
    )(*args)


def kernel(token_ids, embed_weight, proj_weight, scale):
    batch, seq = token_ids.shape
    total = batch * seq
    dim = embed_weight.shape[1]
    n = proj_weight.shape[0]
    scale_f = scale.astype(jnp.float32)
    tokens_flat = token_ids.reshape(-1).astype(jnp.int32)

    nchunks = _NCHUNKS
    mc = total // nchunks
    bm = _BM
    hs = []
    for c in range(nchunks):
        hs.append(_sc_hash_gather(tokens_flat, embed_weight, seq,
                                  c * mc, mc))
    out = None
    for c in range(nchunks):
        out = _tc_project_chunk(hs[c], proj_weight, scale_f, total,
                                c * (mc // bm), out)
    return out.reshape(batch, seq, n)


# speedup vs baseline: 1.1072x; 1.1072x over previous
import functools

import jax
import jax.numpy as jnp
from jax import lax
from jax.experimental import pallas as pl
from jax.experimental.pallas import tpu as pltpu
from jax.experimental.pallas import tpu_sc as plsc

_LANES = 16
_NW = 32
_GCH = 128
_NCHUNKS = 1
_BM = 512
_NBUF = 4


def _sc_hash_gather(tokens_flat, embed_weight, seq_len, chunk_off, mc):
    vocab, dim = embed_weight.shape
    tw = mc // _NW
    nch = tw // _GCH
    mod = jnp.int32(vocab - 1)

    mesh = plsc.VectorSubcoreMesh(core_axis_name="c", subcore_axis_name="s")

    @functools.partial(
        pl.kernel,
        out_type=jax.ShapeDtypeStruct((_NW, nch, _GCH, dim), jnp.float32),
        mesh=mesh,
        scratch_types=[
            pltpu.VMEM((tw + 8,), jnp.int32),
            pltpu.VMEM((nch, _GCH), jnp.int32),
            pltpu.VMEM((nch, _GCH, dim), jnp.float32),
            pltpu.SemaphoreType.DMA((nch,)),
            pltpu.SemaphoreType.DMA,
        ],
    )
    def k(tok_hbm, table_hbm, out_hbm, tok_v, idx_v, rows_v, gsems, wsem):
        wid = lax.axis_index("s") * 2 + lax.axis_index("c")
        base = chunk_off + wid * tw

        if chunk_off == 0:
            @pl.when(wid == 0)
            def _():
                pltpu.sync_copy(tok_hbm.at[pl.ds(0, tw)],
                                tok_v.at[pl.ds(8, tw)])

            @pl.when(wid != 0)
            def _():
                pltpu.sync_copy(tok_hbm.at[pl.ds(base - 8, tw + 8)], tok_v)
        else:
            pltpu.sync_copy(tok_hbm.at[pl.ds(base - 8, tw + 8)], tok_v)

        inv_mod = jnp.float32(1.0 / float(vocab - 1))
        for c in range(nch):
            for j in range(_GCH // _LANES):
                off = c * _GCH + j * _LANES
                prev = tok_v[pl.ds(off + 7, _LANES)]
                cur = tok_v[pl.ds(off + 8, _LANES)]
                x = lax.bitwise_xor(jnp.int32(36313) * cur,
                                    jnp.int32(27191) * prev)
                q = (x.astype(jnp.float32) * inv_mod).astype(jnp.int32)
                r = x - q * mod
                neg = lax.shift_right_arithmetic(r, 31)
                r = r - neg * mod
                t = r - mod
                tneg = lax.shift_right_arithmetic(t, 31)
                r = t - tneg * mod
                idx_v[c, pl.ds(j * _LANES, _LANES)] = r

        @pl.when((base % seq_len) == 0)
        def _():
            first = idx_v[0, pl.ds(0, _LANES)]
            keep = jnp.minimum(lax.iota(jnp.int32, _LANES), 1)
            idx_v[0, pl.ds(0, _LANES)] = first * keep + mod * (1 - keep)

        gathers = [
            pltpu.make_async_copy(table_hbm.at[idx_v.at[c]], rows_v.at[c],
                                  gsems.at[c])
            for c in range(nch)
        ]
        writes = [
            pltpu.make_async_copy(rows_v.at[c], out_hbm.at[wid, c], wsem)
            for c in range(nch)
        ]
        for g in gathers:
            g.start()
        for c in range(nch):
            gathers[c].wait()
            writes[c].start()
        for w in writes:
            w.wait()

    return k(tokens_flat, embed_weight).reshape(mc, dim)


def _tc_project_chunk(h_c, proj_weight, scale, total_m, block_off, prev):
    mc, kdim = h_c.shape
    n = proj_weight.shape[0]
    bm = _BM
    ng = mc // bm
    nbuf = min(_NBUF, ng)
    row_off = block_off * bm

    def body(s_ref, h_ref, w_ref, *rest):
        o_hbm, acc_ref, sems = rest[-3], rest[-2], rest[-1]
        i = pl.program_id(0)
        slot = lax.rem(i, nbuf)

        @pl.when(i >= nbuf)
        def _():
            j = i - nbuf
            pltpu.make_async_copy(
                acc_ref.at[slot],
                o_hbm.at[pl.ds(row_off + j * bm, bm)],
                sems.at[slot]).wait()

        acc = lax.dot_general(
            h_ref[...], w_ref[...], (((1,), (1,)), ((), ())),
            preferred_element_type=jnp.float32)
        acc_ref[slot] = acc * s_ref[0]
        pltpu.make_async_copy(
            acc_ref.at[slot],
            o_hbm.at[pl.ds(row_off + i * bm, bm)],
            sems.at[slot]).start()

        @pl.when(i == ng - 1)
        def _():
            for it in range(ng - nbuf, ng):
                pltpu.make_async_copy(
                    acc_ref.at[it % nbuf],
                    o_hbm.at[pl.ds(row_off + it * bm, bm)],
                    sems.at[it % nbuf]).wait()

    in_specs = [
        pl.BlockSpec(memory_space=pltpu.SMEM),
        pl.BlockSpec((bm, kdim), lambda i: (i, 0)),
        pl.BlockSpec((n, kdim), lambda i: (0, 0)),
    ]
    args = [scale.reshape(1), h_c, proj_weight]
    aliases = {}
    if prev is not None:
        in_specs.append(pl.BlockSpec(memory_space=pl.ANY))
        args.append(prev)
        aliases = {3: 0}

    return pl.pallas_call(
        body,
        grid=(ng,),
        in_specs=in_specs,
        out_specs=pl.BlockSpec(memory_space=pl.ANY),
        out_shape=jax.ShapeDtypeStruct((total_m, n), jnp.float32),
        input_output_aliases=aliases,
        scratch_shapes=[
            pltpu.VMEM((nbuf, bm, n), jnp.float32),
            pltpu.SemaphoreType.DMA((nbuf,)),
        ],
    )(*args)


def kernel(token_ids, embed_weight, proj_weight, scale):
    batch, seq = token_ids.shape
    total = batch * seq
    dim = embed_weight.shape[1]
    n = proj_weight.shape[0]
    scale_f = scale.astype(jnp.float32)
    tokens_flat = token_ids.reshape(-1).astype(jnp.int32)

    nchunks = _NCHUNKS
    mc = total // nchunks
    bm = _BM
    hs = []
    for c in range(nchunks):
        hs.append(_sc_hash_gather(tokens_flat, embed_weight, seq,
                                  c * mc, mc))
    out = None
    for c in range(nchunks):
        out = _tc_project_chunk(hs[c], proj_weight, scale_f, total,
                                c * (mc // bm), out)
    return out.reshape(batch, seq, n)


# speedup vs baseline: 1.1311x; 1.0216x over previous
import functools

import jax
import jax.numpy as jnp
from jax import lax
from jax.experimental import pallas as pl
from jax.experimental.pallas import tpu as pltpu
from jax.experimental.pallas import tpu_sc as plsc

_LANES = 16
_NW = 32
_GCH = 128
_NCHUNKS = 1
_BM = 1024
_NBUF = 4


def _sc_hash_gather(tokens_flat, embed_weight, seq_len, chunk_off, mc):
    vocab, dim = embed_weight.shape
    tw = mc // _NW
    nch = tw // _GCH
    mod = jnp.int32(vocab - 1)

    mesh = plsc.VectorSubcoreMesh(core_axis_name="c", subcore_axis_name="s")

    @functools.partial(
        pl.kernel,
        out_type=jax.ShapeDtypeStruct((_NW, nch, _GCH, dim), jnp.float32),
        mesh=mesh,
        scratch_types=[
            pltpu.VMEM((tw + 8,), jnp.int32),
            pltpu.VMEM((nch, _GCH), jnp.int32),
            pltpu.VMEM((nch, _GCH, dim), jnp.float32),
            pltpu.SemaphoreType.DMA((nch,)),
            pltpu.SemaphoreType.DMA,
        ],
    )
    def k(tok_hbm, table_hbm, out_hbm, tok_v, idx_v, rows_v, gsems, wsem):
        wid = lax.axis_index("s") * 2 + lax.axis_index("c")
        base = chunk_off + wid * tw

        if chunk_off == 0:
            @pl.when(wid == 0)
            def _():
                pltpu.sync_copy(tok_hbm.at[pl.ds(0, tw)],
                                tok_v.at[pl.ds(8, tw)])

            @pl.when(wid != 0)
            def _():
                pltpu.sync_copy(tok_hbm.at[pl.ds(base - 8, tw + 8)], tok_v)
        else:
            pltpu.sync_copy(tok_hbm.at[pl.ds(base - 8, tw + 8)], tok_v)

        inv_mod = jnp.float32(1.0 / float(vocab - 1))
        for c in range(nch):
            for j in range(_GCH // _LANES):
                off = c * _GCH + j * _LANES
                prev = tok_v[pl.ds(off + 7, _LANES)]
                cur = tok_v[pl.ds(off + 8, _LANES)]
                x = lax.bitwise_xor(jnp.int32(36313) * cur,
                                    jnp.int32(27191) * prev)
                q = (x.astype(jnp.float32) * inv_mod).astype(jnp.int32)
                r = x - q * mod
                neg = lax.shift_right_arithmetic(r, 31)
                r = r - neg * mod
                t = r - mod
                tneg = lax.shift_right_arithmetic(t, 31)
                r = t - tneg * mod
                idx_v[c, pl.ds(j * _LANES, _LANES)] = r

        @pl.when((base % seq_len) == 0)
        def _():
            first = idx_v[0, pl.ds(0, _LANES)]
            keep = jnp.minimum(lax.iota(jnp.int32, _LANES), 1)
            idx_v[0, pl.ds(0, _LANES)] = first * keep + mod * (1 - keep)

        gathers = [
            pltpu.make_async_copy(table_hbm.at[idx_v.at[c]], rows_v.at[c],
                                  gsems.at[c])
            for c in range(nch)
        ]
        writes = [
            pltpu.make_async_copy(rows_v.at[c], out_hbm.at[wid, c], wsem)
            for c in range(nch)
        ]
        for g in gathers:
            g.start()
        for c in range(nch):
            gathers[c].wait()
            writes[c].start()
        for w in writes:
            w.wait()

    return k(tokens_flat, embed_weight).reshape(mc, dim)


def _tc_project_chunk(h_c, proj_weight, scale, total_m, block_off, prev):
    mc, kdim = h_c.shape
    n = proj_weight.shape[0]
    bm = _BM
    ng = mc // bm
    nbuf = min(_NBUF, ng)
    row_off = block_off * bm

    def body(s_ref, h_ref, w_ref, *rest):
        o_hbm, acc_ref, sems = rest[-3], rest[-2], rest[-1]
        i = pl.program_id(0)
        slot = lax.rem(i, nbuf)

        @pl.when(i >= nbuf)
        def _():
            j = i - nbuf
            pltpu.make_async_copy(
                acc_ref.at[slot],
                o_hbm.at[pl.ds(row_off + j * bm, bm)],
                sems.at[slot]).wait()

        acc = lax.dot_general(
            h_ref[...], w_ref[...], (((1,), (1,)), ((), ())),
            preferred_element_type=jnp.float32)
        acc_ref[slot] = acc * s_ref[0]
        pltpu.make_async_copy(
            acc_ref.at[slot],
            o_hbm.at[pl.ds(row_off + i * bm, bm)],
            sems.at[slot]).start()

        @pl.when(i == ng - 1)
        def _():
            for it in range(ng - nbuf, ng):
                pltpu.make_async_copy(
                    acc_ref.at[it % nbuf],
                    o_hbm.at[pl.ds(row_off + it * bm, bm)],
                    sems.at[it % nbuf]).wait()

    in_specs = [
        pl.BlockSpec(memory_space=pltpu.SMEM),
        pl.BlockSpec((bm, kdim), lambda i: (i, 0)),
        pl.BlockSpec((n, kdim), lambda i: (0, 0)),
    ]
    args = [scale.reshape(1), h_c, proj_weight]
    aliases = {}
    if prev is not None:
        in_specs.append(pl.BlockSpec(memory_space=pl.ANY))
        args.append(prev)
        aliases = {3: 0}

    return pl.pallas_call(
        body,
        grid=(ng,),
        in_specs=in_specs,
        out_specs=pl.BlockSpec(memory_space=pl.ANY),
        out_shape=jax.ShapeDtypeStruct((total_m, n), jnp.float32),
        input_output_aliases=aliases,
        scratch_shapes=[
            pltpu.VMEM((nbuf, bm, n), jnp.float32),
            pltpu.SemaphoreType.DMA((nbuf,)),
        ],
    )(*args)


def kernel(token_ids, embed_weight, proj_weight, scale):
    batch, seq = token_ids.shape
    total = batch * seq
    dim = embed_weight.shape[1]
    n = proj_weight.shape[0]
    scale_f = scale.astype(jnp.float32)
    tokens_flat = token_ids.reshape(-1).astype(jnp.int32)

    nchunks = _NCHUNKS
    mc = total // nchunks
    bm = _BM
    hs = []
    for c in range(nchunks):
        hs.append(_sc_hash_gather(tokens_flat, embed_weight, seq,
                                  c * mc, mc))
    out = None
    for c in range(nchunks):
        out = _tc_project_chunk(hs[c], proj_weight, scale_f, total,
                                c * (mc // bm), out)
    return out.reshape(batch, seq, n)
